# baseline (speedup 1.0000x reference)
import functools

import jax
import jax.numpy as jnp
from jax import lax
from jax.experimental import pallas as pl
from jax.experimental.pallas import tpu as pltpu
from jax.experimental.pallas import tpu_sc as plsc

B = 16384
DIM = 64
WIDE = 128
NC = 1
NS = 16
NW = NC * NS
BPW = B // NW
CHUNK = 128
NCHUNK = BPW // CHUNK


def _body(users, pos_items, neg_items, user_table, item_table,
          pos_out, neg_out,
          idx_u, idx_p, idx_n, u_rows, p_rows, n_rows,
          pos_v, neg_v, sem0, sem1):
    wid = lax.axis_index("s") * NC + lax.axis_index("c")
    base = wid * BPW

    for j in range(NCHUNK):
        off = base + j * CHUNK
        pltpu.sync_copy(users.at[pl.ds(off, CHUNK)], idx_u.at[j])
        pltpu.sync_copy(pos_items.at[pl.ds(off, CHUNK)], idx_p.at[j])
        pltpu.sync_copy(neg_items.at[pl.ds(off, CHUNK)], idx_n.at[j])

    sems = (sem0, sem1)

    def chunk_copies(j):
        buf = j % 2
        sem = sems[buf]
        return (
            pltpu.make_async_copy(user_table.at[idx_u.at[j]], u_rows.at[buf], sem),
            pltpu.make_async_copy(item_table.at[idx_p.at[j]], p_rows.at[buf], sem),
            pltpu.make_async_copy(item_table.at[idx_n.at[j]], n_rows.at[buf], sem),
        )

    lane = lax.iota(jnp.int32, 16)
    perms = [lane ^ s for s in (8, 4, 2, 1)]
    gdn = lax.GatherDimensionNumbers(
        offset_dims=(), collapsed_slice_dims=(0,), start_index_map=(0,))

    def shuffle(v, p):
        return lax.gather(v, p[:, None], gdn, slice_sizes=(1,),
                          mode=lax.GatherScatterMode.PROMISE_IN_BOUNDS)

    def lanesum(v):
        for p in perms:
            v = v + shuffle(v, p)
        return v

    def compute_chunk(j):
        buf = j % 2
        ur, pr, nr = u_rows.at[buf], p_rows.at[buf], n_rows.at[buf]

        def group(g, _):
            r0 = g * 16
            accp_v = jnp.zeros((16,), jnp.float32)
            accn_v = jnp.zeros((16,), jnp.float32)
            for r in range(16):
                ri = r0 + r
                u0 = ur[ri, pl.ds(0, 16)]
                u1 = ur[ri, pl.ds(16, 16)]
                u2 = ur[ri, pl.ds(32, 16)]
                u3 = ur[ri, pl.ds(48, 16)]
                p0 = pr[ri, pl.ds(0, 16)]
                p1 = pr[ri, pl.ds(16, 16)]
                p2 = pr[ri, pl.ds(32, 16)]
                p3 = pr[ri, pl.ds(48, 16)]
                n0 = nr[ri, pl.ds(0, 16)]
                n1 = nr[ri, pl.ds(16, 16)]
                n2 = nr[ri, pl.ds(32, 16)]
                n3 = nr[ri, pl.ds(48, 16)]
                accp = (u0 * p0 + u1 * p1) + (u2 * p2 + u3 * p3)
                accn = (u0 * n0 + u1 * n1) + (u2 * n2 + u3 * n3)
                sel = lane == r
                accp_v = jnp.where(sel, lanesum(accp), accp_v)
                accn_v = jnp.where(sel, lanesum(accn), accn_v)
            out = pl.ds(j * CHUNK + r0, 16)
            pos_v[out] = accp_v
            neg_v[out] = accn_v
            return 0

        lax.fori_loop(0, CHUNK // 16, group, 0)

    for c in chunk_copies(0):
        c.start()
    for j in range(NCHUNK):
        if j + 1 < NCHUNK:
            for c in chunk_copies(j + 1):
                c.start()
        for c in chunk_copies(j):
            c.wait()
        compute_chunk(j)

    pltpu.sync_copy(pos_v, pos_out.at[pl.ds(base, BPW)])
    pltpu.sync_copy(neg_v, neg_out.at[pl.ds(base, BPW)])


@functools.partial(jax.jit, donate_argnums=())
def _run(users, pos_items, neg_items, user_table, item_table):
    mesh = plsc.VectorSubcoreMesh(core_axis_name="c", subcore_axis_name="s", num_cores=1)
    f = pl.kernel(
        _body,
        out_type=(
            jax.ShapeDtypeStruct((B,), jnp.float32),
            jax.ShapeDtypeStruct((B,), jnp.float32),
        ),
        mesh=mesh,
        scratch_types=[
            pltpu.VMEM((NCHUNK, CHUNK), jnp.int32),
            pltpu.VMEM((NCHUNK, CHUNK), jnp.int32),
            pltpu.VMEM((NCHUNK, CHUNK), jnp.int32),
            pltpu.VMEM((2, CHUNK, WIDE), jnp.float32),
            pltpu.VMEM((2, CHUNK, WIDE), jnp.float32),
            pltpu.VMEM((2, CHUNK, WIDE), jnp.float32),
            pltpu.VMEM((BPW,), jnp.float32),
            pltpu.VMEM((BPW,), jnp.float32),
            pltpu.SemaphoreType.DMA,
            pltpu.SemaphoreType.DMA,
        ],
        name="basemf_sc_scores",
    )
    zeros = jnp.zeros((B, WIDE - DIM), jnp.float32)
    z_u = jnp.broadcast_to(zeros[:1], (user_table.shape[0], WIDE - DIM))
    pad_u = jnp.concatenate([user_table, z_u], axis=1)
    pad_i = jnp.concatenate([item_table, z_u], axis=1)
    return f(users, pos_items, neg_items, pad_u, pad_i)


def kernel(users, pos_items, neg_items, user_table, item_table):
    return _run(users, pos_items, neg_items, user_table, item_table)


# speedup vs baseline: 1.0179x; 1.0179x over previous
import functools

import jax
import jax.numpy as jnp
from jax import lax
from jax.experimental import pallas as pl
from jax.experimental.pallas import tpu as pltpu
from jax.experimental.pallas import tpu_sc as plsc

B = 16384
DIM = 64
WIDE = 128
NC = 2
NS = 16
NW = NC * NS
BPW = B // NW
CHUNK = 128
NCHUNK = BPW // CHUNK


def _body(users, pos_items, neg_items, user_table, item_table,
          pos_out, neg_out,
          idx_u, idx_p, idx_n, u_rows, p_rows, n_rows,
          pos_v, neg_v, sem0, sem1):
    wid = lax.axis_index("s") * NC + lax.axis_index("c")
    base = wid * BPW

    for j in range(NCHUNK):
        off = base + j * CHUNK
        pltpu.sync_copy(users.at[pl.ds(off, CHUNK)], idx_u.at[j])
        pltpu.sync_copy(pos_items.at[pl.ds(off, CHUNK)], idx_p.at[j])
        pltpu.sync_copy(neg_items.at[pl.ds(off, CHUNK)], idx_n.at[j])

    sems = (sem0, sem1)

    def chunk_copies(j):
        buf = j % 2
        sem = sems[buf]
        return (
            pltpu.make_async_copy(user_table.at[idx_u.at[j]], u_rows.at[buf], sem),
            pltpu.make_async_copy(item_table.at[idx_p.at[j]], p_rows.at[buf], sem),
            pltpu.make_async_copy(item_table.at[idx_n.at[j]], n_rows.at[buf], sem),
        )

    lane = lax.iota(jnp.int32, 16)
    perms = [lane ^ s for s in (8, 4, 2, 1)]
    gdn = lax.GatherDimensionNumbers(
        offset_dims=(), collapsed_slice_dims=(0,), start_index_map=(0,))

    def shuffle(v, p):
        return lax.gather(v, p[:, None], gdn, slice_sizes=(1,),
                          mode=lax.GatherScatterMode.PROMISE_IN_BOUNDS)

    def lanesum(v):
        for p in perms:
            v = v + shuffle(v, p)
        return v

    def compute_chunk(j):
        buf = j % 2
        ur, pr, nr = u_rows.at[buf], p_rows.at[buf], n_rows.at[buf]

        def group(g, _):
            r0 = g * 16
            accp_v = jnp.zeros((16,), jnp.float32)
            accn_v = jnp.zeros((16,), jnp.float32)
            for r in range(16):
                ri = r0 + r
                u0 = ur[ri, pl.ds(0, 16)]
                u1 = ur[ri, pl.ds(16, 16)]
                u2 = ur[ri, pl.ds(32, 16)]
                u3 = ur[ri, pl.ds(48, 16)]
                p0 = pr[ri, pl.ds(0, 16)]
                p1 = pr[ri, pl.ds(16, 16)]
                p2 = pr[ri, pl.ds(32, 16)]
                p3 = pr[ri, pl.ds(48, 16)]
                n0 = nr[ri, pl.ds(0, 16)]
                n1 = nr[ri, pl.ds(16, 16)]
                n2 = nr[ri, pl.ds(32, 16)]
                n3 = nr[ri, pl.ds(48, 16)]
                accp = (u0 * p0 + u1 * p1) + (u2 * p2 + u3 * p3)
                accn = (u0 * n0 + u1 * n1) + (u2 * n2 + u3 * n3)
                sel = lane == r
                accp_v = jnp.where(sel, lanesum(accp), accp_v)
                accn_v = jnp.where(sel, lanesum(accn), accn_v)
            out = pl.ds(j * CHUNK + r0, 16)
            pos_v[out] = accp_v
            neg_v[out] = accn_v
            return 0

        lax.fori_loop(0, CHUNK // 16, group, 0)

    for c in chunk_copies(0):
        c.start()
    for j in range(NCHUNK):
        if j + 1 < NCHUNK:
            for c in chunk_copies(j + 1):
                c.start()
        for c in chunk_copies(j):
            c.wait()
        compute_chunk(j)

    pltpu.sync_copy(pos_v, pos_out.at[pl.ds(base, BPW)])
    pltpu.sync_copy(neg_v, neg_out.at[pl.ds(base, BPW)])


@functools.partial(jax.jit, donate_argnums=())
def _run(users, pos_items, neg_items, user_table, item_table):
    mesh = plsc.VectorSubcoreMesh(core_axis_name="c", subcore_axis_name="s")
    f = pl.kernel(
        _body,
        out_type=(
            jax.ShapeDtypeStruct((B,), jnp.float32),
            jax.ShapeDtypeStruct((B,), jnp.float32),
        ),
        mesh=mesh,
        scratch_types=[
            pltpu.VMEM((NCHUNK, CHUNK), jnp.int32),
            pltpu.VMEM((NCHUNK, CHUNK), jnp.int32),
            pltpu.VMEM((NCHUNK, CHUNK), jnp.int32),
            pltpu.VMEM((2, CHUNK, WIDE), jnp.float32),
            pltpu.VMEM((2, CHUNK, WIDE), jnp.float32),
            pltpu.VMEM((2, CHUNK, WIDE), jnp.float32),
            pltpu.VMEM((BPW,), jnp.float32),
            pltpu.VMEM((BPW,), jnp.float32),
            pltpu.SemaphoreType.DMA,
            pltpu.SemaphoreType.DMA,
        ],
        name="basemf_sc_scores",
    )
    pad_u = jnp.pad(user_table, ((0, 0), (0, WIDE - DIM)))
    pad_i = jnp.pad(item_table, ((0, 0), (0, WIDE - DIM)))
    return f(users, pos_items, neg_items, pad_u, pad_i)


def kernel(users, pos_items, neg_items, user_table, item_table):
    return _run(users, pos_items, neg_items, user_table, item_table)


# speedup vs baseline: 1.0213x; 1.0033x over previous
import functools

import jax
import jax.numpy as jnp
from jax import lax
from jax.experimental import pallas as pl
from jax.experimental.pallas import tpu as pltpu
from jax.experimental.pallas import tpu_sc as plsc

B = 16384
DIM = 64
WIDE = 128
NC = 2
NS = 16
NW = NC * NS
BPW = B // NW
CHUNK = 128
NCHUNK = BPW // CHUNK


def _gather_users_body(users, user_table, u_out, idx_u, rows_v, sem):
    wid = lax.axis_index("s") * NC + lax.axis_index("c")
    base = wid * BPW

    for j in range(NCHUNK):
        pltpu.sync_copy(users.at[pl.ds(base + j * CHUNK, CHUNK)], idx_u.at[j])

    copies = [
        pltpu.make_async_copy(
            user_table.at[idx_u.at[j]],
            rows_v.at[pl.ds(j * CHUNK, CHUNK)], sem)
        for j in range(NCHUNK)
    ]
    for c in copies:
        c.start()
    for c in copies:
        c.wait()

    pltpu.sync_copy(rows_v, u_out.at[pl.ds(base, BPW)])


def _scores_body(pos_items, neg_items, item_table, u_e,
                 pos_out, neg_out,
                 idx_p, idx_n, u_rows, p_rows, n_rows,
                 pos_v, neg_v, sem0, sem1):
    wid = lax.axis_index("s") * NC + lax.axis_index("c")
    base = wid * BPW

    for j in range(NCHUNK):
        off = base + j * CHUNK
        pltpu.sync_copy(pos_items.at[pl.ds(off, CHUNK)], idx_p.at[j])
        pltpu.sync_copy(neg_items.at[pl.ds(off, CHUNK)], idx_n.at[j])

    sems = (sem0, sem1)

    def chunk_copies(j):
        buf = j % 2
        sem = sems[buf]
        return (
            pltpu.make_async_copy(
                u_e.at[pl.ds(base + j * CHUNK, CHUNK)], u_rows.at[buf], sem),
            pltpu.make_async_copy(
                item_table.at[idx_p.at[j]], p_rows.at[buf], sem),
            pltpu.make_async_copy(
                item_table.at[idx_n.at[j]], n_rows.at[buf], sem),
        )

    lane = lax.iota(jnp.int32, 16)
    perms = [lane ^ s for s in (8, 4, 2, 1)]
    gdn = lax.GatherDimensionNumbers(
        offset_dims=(), collapsed_slice_dims=(0,), start_index_map=(0,))

    def shuffle(v, p):
        return lax.gather(v, p[:, None], gdn, slice_sizes=(1,),
                          mode=lax.GatherScatterMode.PROMISE_IN_BOUNDS)

    def lanesum(v):
        for p in perms:
            v = v + shuffle(v, p)
        return v

    def compute_chunk(j):
        buf = j % 2
        ur, pr, nr = u_rows.at[buf], p_rows.at[buf], n_rows.at[buf]

        def group(g, _):
            r0 = g * 16
            accp_v = jnp.zeros((16,), jnp.float32)
            accn_v = jnp.zeros((16,), jnp.float32)
            for r in range(16):
                ri = r0 + r
                u0 = ur[ri, pl.ds(0, 16)]
                u1 = ur[ri, pl.ds(16, 16)]
                u2 = ur[ri, pl.ds(32, 16)]
                u3 = ur[ri, pl.ds(48, 16)]
                p0 = pr[ri, pl.ds(0, 16)]
                p1 = pr[ri, pl.ds(16, 16)]
                p2 = pr[ri, pl.ds(32, 16)]
                p3 = pr[ri, pl.ds(48, 16)]
                n0 = nr[ri, pl.ds(0, 16)]
                n1 = nr[ri, pl.ds(16, 16)]
                n2 = nr[ri, pl.ds(32, 16)]
                n3 = nr[ri, pl.ds(48, 16)]
                accp = (u0 * p0 + u1 * p1) + (u2 * p2 + u3 * p3)
                accn = (u0 * n0 + u1 * n1) + (u2 * n2 + u3 * n3)
                sel = lane == r
                accp_v = jnp.where(sel, lanesum(accp), accp_v)
                accn_v = jnp.where(sel, lanesum(accn), accn_v)
            out = pl.ds(j * CHUNK + r0, 16)
            pos_v[out] = accp_v
            neg_v[out] = accn_v
            return 0

        lax.fori_loop(0, CHUNK // 16, group, 0)

    for c in chunk_copies(0):
        c.start()
    for j in range(NCHUNK):
        if j + 1 < NCHUNK:
            for c in chunk_copies(j + 1):
                c.start()
        for c in chunk_copies(j):
            c.wait()
        compute_chunk(j)

    pltpu.sync_copy(pos_v, pos_out.at[pl.ds(base, BPW)])
    pltpu.sync_copy(neg_v, neg_out.at[pl.ds(base, BPW)])


@functools.partial(jax.jit, donate_argnums=())
def _run(users, pos_items, neg_items, user_table, item_table):
    mesh = plsc.VectorSubcoreMesh(core_axis_name="c", subcore_axis_name="s")
    f1 = pl.kernel(
        _gather_users_body,
        out_type=jax.ShapeDtypeStruct((B, WIDE), jnp.float32),
        mesh=mesh,
        scratch_types=[
            pltpu.VMEM((NCHUNK, CHUNK), jnp.int32),
            pltpu.VMEM((BPW, WIDE), jnp.float32),
            pltpu.SemaphoreType.DMA,
        ],
        name="basemf_sc_user_gather",
    )
    f2 = pl.kernel(
        _scores_body,
        out_type=(
            jax.ShapeDtypeStruct((B,), jnp.float32),
            jax.ShapeDtypeStruct((B,), jnp.float32),
        ),
        mesh=mesh,
        scratch_types=[
            pltpu.VMEM((NCHUNK, CHUNK), jnp.int32),
            pltpu.VMEM((NCHUNK, CHUNK), jnp.int32),
            pltpu.VMEM((2, CHUNK, WIDE), jnp.float32),
            pltpu.VMEM((2, CHUNK, WIDE), jnp.float32),
            pltpu.VMEM((2, CHUNK, WIDE), jnp.float32),
            pltpu.VMEM((BPW,), jnp.float32),
            pltpu.VMEM((BPW,), jnp.float32),
            pltpu.SemaphoreType.DMA,
            pltpu.SemaphoreType.DMA,
        ],
        name="basemf_sc_scores",
    )
    pad_u = jnp.pad(user_table, ((0, 0), (0, WIDE - DIM)))
    pad_i = jnp.pad(item_table, ((0, 0), (0, WIDE - DIM)))
    u_e = f1(users, pad_u)
    return f2(pos_items, neg_items, pad_i, u_e)


def kernel(users, pos_items, neg_items, user_table, item_table):
    return _run(users, pos_items, neg_items, user_table, item_table)
